# baseline (speedup 1.0000x reference)
import functools

import jax
import jax.numpy as jnp
from jax import lax
from jax.experimental import pallas as pl
from jax.experimental.pallas import tpu as pltpu
from jax.experimental.pallas import tpu_sc as plsc

N = 2048
C = 1024
K = 20
HALF = N // 2
NUM_WORKERS = 32
ROWS_PER_WORKER = HALF // NUM_WORKERS
CHUNK = 16
NCHUNK = ROWS_PER_WORKER // CHUNK
COLBLK = 128
TC_BLK = 512


def _sc_body(base, logits_hbm, relsT_hbm, maskT_hbm, t_hbm, m_hbm,
             log_v0, log_v1, rels_v, mask_v, tv, mv, sem0, sem1, sem2, sem3):
    wid = lax.axis_index("s") * 2 + lax.axis_index("c")
    obase = wid * ROWS_PER_WORKER
    tbase = base + obase
    c128 = (tbase // COLBLK) * COLBLK
    coff = tbase - c128
    logs = [log_v0, log_v1]
    sems = [sem0, sem1]
    cps = [None, None]
    cps[0] = pltpu.async_copy(
        logits_hbm.at[pl.ds(tbase, CHUNK), :], logs[0], sems[0])
    cp_r = pltpu.async_copy(relsT_hbm.at[:, pl.ds(c128, COLBLK)], rels_v, sem2)
    cp_m = pltpu.async_copy(maskT_hbm.at[:, pl.ds(c128, COLBLK)], mask_v, sem3)
    cp_r.wait()
    cp_m.wait()
    for ci in range(NCHUNK):
        if ci + 1 < NCHUNK:
            nb = (ci + 1) % 2
            cps[nb] = pltpu.async_copy(
                logits_hbm.at[pl.ds(tbase + (ci + 1) * CHUNK, CHUNK), :],
                logs[nb], sems[nb])
        cps[ci % 2].wait()
        log_v = logs[ci % 2]
        rows = lax.iota(jnp.int32, CHUNK)
        best_s = jnp.full((CHUNK,), -1.0, jnp.float32)
        best_g = jnp.zeros((CHUNK,), jnp.float32)
        best_m = jnp.zeros((CHUNK,), jnp.float32)

        def body(k, carry):
            bs, bg, bm = carry
            rk = rels_v[k, pl.ds(coff + ci * CHUNK, CHUNK)]
            mk = mask_v[k, pl.ds(coff + ci * CHUNK, CHUNK)]
            g = plsc.load_gather(log_v, [rows, rk])
            s = mk / (1.0 + jnp.exp(-g))
            upd = s > bs
            return (jnp.where(upd, s, bs), jnp.where(upd, g, bg),
                    jnp.where(upd, mk, bm))
        best_s, best_g, best_m = lax.fori_loop(
            0, K, body, (best_s, best_g, best_m))
        zz = jnp.zeros((CHUNK,), jnp.int32)
        plsc.store_scatter(tv, [ci * CHUNK + rows, zz], best_g)
        plsc.store_scatter(mv, [ci * CHUNK + rows, zz], best_m)
    pltpu.sync_copy(tv, t_hbm.at[pl.ds(obase, ROWS_PER_WORKER), :])
    pltpu.sync_copy(mv, m_hbm.at[pl.ds(obase, ROWS_PER_WORKER), :])


def _make_sc(base):
    return functools.partial(
        pl.kernel,
        out_type=(jax.ShapeDtypeStruct((HALF, 1), jnp.float32),
                  jax.ShapeDtypeStruct((HALF, 1), jnp.float32)),
        mesh=plsc.VectorSubcoreMesh(core_axis_name="c", subcore_axis_name="s"),
        compiler_params=pltpu.CompilerParams(needs_layout_passes=False),
        scratch_types=[
            pltpu.VMEM((CHUNK, C), jnp.float32),
            pltpu.VMEM((CHUNK, C), jnp.float32),
            pltpu.VMEM((K, COLBLK), jnp.int32),
            pltpu.VMEM((K, COLBLK), jnp.float32),
            pltpu.VMEM((ROWS_PER_WORKER, 1), jnp.float32),
            pltpu.VMEM((ROWS_PER_WORKER, 1), jnp.float32),
            pltpu.SemaphoreType.DMA,
            pltpu.SemaphoreType.DMA,
            pltpu.SemaphoreType.DMA,
            pltpu.SemaphoreType.DMA,
        ],
    )(functools.partial(_sc_body, base))


_sc_lo = _make_sc(0)
_sc_hi = _make_sc(HALF)


def _tc_body(logits_ref, htr_ref, t_ref, m_ref, out_ref, acc_ref):
    i = pl.program_id(0)

    @pl.when(i == 0)
    def _init():
        acc_ref[0] = 0.0
        acc_ref[1] = 0.0

    lg = logits_ref[...]
    cmp = (lg >= t_ref[...]).astype(jnp.float32)
    rank = jnp.sum(cmp, axis=1, keepdims=True)
    neg = jnp.sum(cmp * (1.0 - htr_ref[...]), axis=1, keepdims=True)
    neg = jnp.maximum(neg, 0.1)
    mcol = m_ref[...]
    acc_ref[0] += jnp.sum(mcol * neg / rank)
    acc_ref[1] += jnp.sum(mcol)

    @pl.when(i == (HALF // TC_BLK) - 1)
    def _fin():
        out_ref[0, 0] = acc_ref[0]
        out_ref[0, 1] = acc_ref[1]


def _make_tc(base_blk):
    return pl.pallas_call(
        _tc_body,
        grid=(HALF // TC_BLK,),
        in_specs=[
            pl.BlockSpec((TC_BLK, C), lambda i: (base_blk + i, 0)),
            pl.BlockSpec((TC_BLK, C), lambda i: (base_blk + i, 0)),
            pl.BlockSpec((TC_BLK, 1), lambda i: (i, 0)),
            pl.BlockSpec((TC_BLK, 1), lambda i: (i, 0)),
        ],
        out_specs=pl.BlockSpec(memory_space=pltpu.SMEM),
        out_shape=jax.ShapeDtypeStruct((1, 2), jnp.float32),
        scratch_shapes=[pltpu.SMEM((2,), jnp.float32)],
    )


_tc_lo = _make_tc(0)
_tc_hi = _make_tc(HALF // TC_BLK)


def kernel(logits, Match_htr, match_rels, match_rels_mask):
    relsT = match_rels.astype(jnp.int32).T
    maskT = match_rels_mask.astype(jnp.float32).T
    t0, m0 = _sc_lo(logits, relsT, maskT)
    t1, m1 = _sc_hi(logits, relsT, maskT)
    p0 = _tc_lo(logits, Match_htr, t0, m0)
    p1 = _tc_hi(logits, Match_htr, t1, m1)
    return (p0[0, 0] + p1[0, 0]) / (p0[0, 1] + p1[0, 1])


# speedup vs baseline: 1.2088x; 1.2088x over previous
import functools

import jax
import jax.numpy as jnp
from jax import lax
from jax.experimental import pallas as pl
from jax.experimental.pallas import tpu as pltpu
from jax.experimental.pallas import tpu_sc as plsc

N = 2048
C = 1024
K = 20
NUM_WORKERS = 32
ROWS_PER_WORKER = N // NUM_WORKERS
CHUNK = 16
NCHUNK = ROWS_PER_WORKER // CHUNK
COLBLK = 128
TC_BLK = 1024


def _sc_top1(logits_hbm, relsT_hbm, maskT_hbm, t_hbm, m_hbm,
             log_v0, log_v1, rels_v, mask_v, tv, mv, sem0, sem1, sem2, sem3):
    wid = lax.axis_index("s") * 2 + lax.axis_index("c")
    tbase = wid * ROWS_PER_WORKER
    c128 = (tbase // COLBLK) * COLBLK
    coff = tbase - c128
    logs = [log_v0, log_v1]
    sems = [sem0, sem1]
    cps = [None, None]
    cps[0] = pltpu.async_copy(
        logits_hbm.at[pl.ds(tbase, CHUNK), :], logs[0], sems[0])
    cp_r = pltpu.async_copy(relsT_hbm.at[:, pl.ds(c128, COLBLK)], rels_v, sem2)
    cp_m = pltpu.async_copy(maskT_hbm.at[:, pl.ds(c128, COLBLK)], mask_v, sem3)
    cp_r.wait()
    cp_m.wait()
    for ci in range(NCHUNK):
        if ci + 1 < NCHUNK:
            nb = (ci + 1) % 2
            cps[nb] = pltpu.async_copy(
                logits_hbm.at[pl.ds(tbase + (ci + 1) * CHUNK, CHUNK), :],
                logs[nb], sems[nb])
        cps[ci % 2].wait()
        log_v = logs[ci % 2]
        rows = lax.iota(jnp.int32, CHUNK)
        best_s = jnp.full((CHUNK,), -1.0, jnp.float32)
        best_g = jnp.zeros((CHUNK,), jnp.float32)
        best_m = jnp.zeros((CHUNK,), jnp.float32)

        def body(k, carry):
            bs, bg, bm = carry
            rk = rels_v[k, pl.ds(coff + ci * CHUNK, CHUNK)]
            mk = mask_v[k, pl.ds(coff + ci * CHUNK, CHUNK)]
            g = plsc.load_gather(log_v, [rows, rk])
            s = mk / (1.0 + jnp.exp(-g))
            upd = s > bs
            return (jnp.where(upd, s, bs), jnp.where(upd, g, bg),
                    jnp.where(upd, mk, bm))
        best_s, best_g, best_m = lax.fori_loop(
            0, K, body, (best_s, best_g, best_m))
        zz = jnp.zeros((CHUNK,), jnp.int32)
        plsc.store_scatter(tv, [ci * CHUNK + rows, zz], best_g)
        plsc.store_scatter(mv, [ci * CHUNK + rows, zz], best_m)
    pltpu.sync_copy(tv, t_hbm.at[pl.ds(tbase, ROWS_PER_WORKER), :])
    pltpu.sync_copy(mv, m_hbm.at[pl.ds(tbase, ROWS_PER_WORKER), :])


_sc_call = functools.partial(
    pl.kernel,
    out_type=(jax.ShapeDtypeStruct((N, 1), jnp.float32),
              jax.ShapeDtypeStruct((N, 1), jnp.float32)),
    mesh=plsc.VectorSubcoreMesh(core_axis_name="c", subcore_axis_name="s"),
    compiler_params=pltpu.CompilerParams(needs_layout_passes=False),
    scratch_types=[
        pltpu.VMEM((CHUNK, C), jnp.float32),
        pltpu.VMEM((CHUNK, C), jnp.float32),
        pltpu.VMEM((K, COLBLK), jnp.int32),
        pltpu.VMEM((K, COLBLK), jnp.float32),
        pltpu.VMEM((ROWS_PER_WORKER, 1), jnp.float32),
        pltpu.VMEM((ROWS_PER_WORKER, 1), jnp.float32),
        pltpu.SemaphoreType.DMA,
        pltpu.SemaphoreType.DMA,
        pltpu.SemaphoreType.DMA,
        pltpu.SemaphoreType.DMA,
    ],
)(_sc_top1)


def _tc_loss_body(logits_ref, htr_ref, t_ref, m_ref, out_ref, acc_ref):
    i = pl.program_id(0)

    @pl.when(i == 0)
    def _init():
        acc_ref[0] = 0.0
        acc_ref[1] = 0.0

    lg = logits_ref[...]
    cmp = (lg >= t_ref[...]).astype(jnp.float32)
    rank = jnp.sum(cmp, axis=1, keepdims=True)
    neg = jnp.sum(cmp * (1.0 - htr_ref[...]), axis=1, keepdims=True)
    neg = jnp.maximum(neg, 0.1)
    mcol = m_ref[...]
    acc_ref[0] += jnp.sum(mcol * neg / rank)
    acc_ref[1] += jnp.sum(mcol)

    @pl.when(i == (N // TC_BLK) - 1)
    def _fin():
        out_ref[0, 0] = acc_ref[0] / acc_ref[1]


_tc_call = pl.pallas_call(
    _tc_loss_body,
    grid=(N // TC_BLK,),
    in_specs=[
        pl.BlockSpec((TC_BLK, C), lambda i: (i, 0)),
        pl.BlockSpec((TC_BLK, C), lambda i: (i, 0)),
        pl.BlockSpec((TC_BLK, 1), lambda i: (i, 0)),
        pl.BlockSpec((TC_BLK, 1), lambda i: (i, 0)),
    ],
    out_specs=pl.BlockSpec(memory_space=pltpu.SMEM),
    out_shape=jax.ShapeDtypeStruct((1, 1), jnp.float32),
    scratch_shapes=[pltpu.SMEM((2,), jnp.float32)],
)


def kernel(logits, Match_htr, match_rels, match_rels_mask):
    t, m = _sc_call(logits, match_rels.astype(jnp.int32).T,
                    match_rels_mask.astype(jnp.float32).T)
    loss = _tc_call(logits, Match_htr, t, m)
    return loss[0, 0]
